# baseline (speedup 1.0000x reference)
import functools
import jax
import jax.numpy as jnp
from jax import lax
from jax.experimental import pallas as pl
from jax.experimental.pallas import tpu as pltpu
from jax.experimental.pallas import tpu_sc as plsc

N1 = 10000
N2 = 199
N2P = 256
E2 = 3184
F = 128
B = 16
K = 50
D2 = 64

_INTERPRET = False

E1 = 640000
NC = 2
NS = 16
EC = 100
RPT = E1 // (NC * NS * EC)


def _sc_hist(dst_flat):
    mesh = plsc.VectorSubcoreMesh(core_axis_name="c", subcore_axis_name="s")
    EPT = E1 // (NC * NS)

    @functools.partial(
        pl.kernel, mesh=mesh,
        out_type=jax.ShapeDtypeStruct((NC * NS, N1), jnp.float32),
        scratch_types=[pltpu.VMEM((EPT,), jnp.int32),
                       pltpu.VMEM((N1,), jnp.float32)],
        compiler_params=pltpu.CompilerParams(needs_layout_passes=False))
    def k(df_h, hist_h, df_v, hist_v):
        c = lax.axis_index("c")
        s = lax.axis_index("s")
        w = c * NS + s
        pltpu.sync_copy(df_h.at[pl.ds(w * EPT, EPT)], df_v)
        zv = jnp.zeros((16,), jnp.float32)

        def zbody(i, carry):
            hist_v[pl.ds(i * 16, 16)] = zv
            return carry
        lax.fori_loop(0, N1 // 16, zbody, 0)

        def hbody(r, carry):
            d = df_v[pl.ds(r * 16, 16)]
            counts, islast = plsc.scan_count(d)
            plsc.addupdate_scatter(hist_v, [d], counts.astype(jnp.float32),
                                   mask=islast)
            return carry
        lax.fori_loop(0, EPT // 16, hbody, 0)
        pltpu.sync_copy(hist_v, hist_h.at[w])

    return k(dst_flat)


KP = 64
NEGF = -3.0e38
BIGI = 1 << 30


def _sc_topk(keys, batch, starts, counts, out1p):
    mesh = plsc.VectorSubcoreMesh(core_axis_name="c", subcore_axis_name="s")
    NV = N1 // 16

    @functools.partial(
        pl.kernel, mesh=mesh,
        out_type=jax.ShapeDtypeStruct((KP * B, F), jnp.float32),
        scratch_types=[
            pltpu.VMEM((N1,), jnp.float32),
            pltpu.VMEM((N1,), jnp.int32),
            pltpu.VMEM((16,), jnp.int32),
            pltpu.VMEM((16,), jnp.int32),
            pltpu.VMEM((KP,), jnp.int32),
            pltpu.VMEM((KP,), jnp.int32),
            pltpu.VMEM((KP, F), jnp.float32),
            pltpu.SemaphoreType.DMA,
            pltpu.SemaphoreType.DMA,
        ],
        compiler_params=pltpu.CompilerParams(needs_layout_passes=False),
    )
    def k(keys_h, batch_h, starts_h, counts_h, o1_h, sel_h,
          kb, bb, st_v, ct_v, win, ridx, rows, g0, s0):
        c = lax.axis_index("c")
        s = lax.axis_index("s")

        @pl.when(c >= 0)
        def _():
            pltpu.sync_copy(keys_h, kb)
            pltpu.sync_copy(batch_h, bb)
            pltpu.sync_copy(starts_h, st_v)
            pltpu.sync_copy(counts_h, ct_v)
            iota = lax.iota(jnp.int32, 16)
            lane0 = iota == 0
            svec = jnp.full((16,), s, jnp.int32)
            dnums = lax.GatherDimensionNumbers(
                offset_dims=(), collapsed_slice_dims=(0,),
                start_index_map=(0,))
            start_s = jnp.max(lax.gather(
                st_v[...], svec[:, None], dnums, (1,),
                mode=lax.GatherScatterMode.PROMISE_IN_BOUNDS))
            cnt_s = jnp.max(lax.gather(
                ct_v[...], svec[:, None], dnums, (1,),
                mode=lax.GatherScatterMode.PROMISE_IN_BOUNDS))
            jlo = start_s // 16
            jhi = (start_s + cnt_s + 15) // 16

            def mbody(j, carry):
                kv = kb[pl.ds(j * 16, 16)]
                bv = bb[pl.ds(j * 16, 16)]
                kb[pl.ds(j * 16, 16)] = jnp.where(bv == s, kv, NEGF)
                return carry
            lax.fori_loop(jlo, jhi, mbody, 0)

            for q in range(KP // 16):
                win[pl.ds(q * 16, 16)] = jnp.zeros((16,), jnp.int32)
                ridx[pl.ds(q * 16, 16)] = (iota + q * 16) * B + s

            def extract(kk, carry):
                def p1(j, m):
                    return jnp.maximum(m, kb[pl.ds(j * 16, 16)])
                mk = jnp.max(lax.fori_loop(
                    jlo, jhi, p1, jnp.full((16,), NEGF, jnp.float32)))

                def p2(j, mi):
                    kv = kb[pl.ds(j * 16, 16)]
                    cand = jnp.where(kv == mk, iota + j * 16, BIGI)
                    return jnp.minimum(mi, cand)
                wi = jnp.min(lax.fori_loop(
                    jlo, jhi, p2, jnp.full((16,), BIGI, jnp.int32)))
                wi = jnp.minimum(wi, N1 - 1)
                wv = jnp.full((16,), wi, jnp.int32)
                plsc.store_scatter(kb, [wv],
                                   jnp.full((16,), NEGF, jnp.float32),
                                   mask=lane0)
                plsc.store_scatter(win, [jnp.full((16,), kk, jnp.int32)],
                                   wv, mask=lane0)
                return carry
            lax.fori_loop(0, K, extract, 0)

            pltpu.async_copy(o1_h.at[win], rows, g0).wait()
            pltpu.async_copy(rows, sel_h.at[ridx], s0).wait()

    return k(keys, batch, starts, counts, out1p)


def _tc_a_body(x1, msum1, histsT, Wl1, bl1, Wr1, Wl2, bl2, Wr2, pb_o, cnt_o):
    cnt = jnp.maximum(jnp.sum(histsT[...], axis=1, keepdims=True), 1.0)
    mean = msum1[...] / cnt
    h = jnp.dot(mean, Wl1[...], preferred_element_type=jnp.float32)
    h = h + bl1[...] + jnp.dot(x1[...], Wr1[...], preferred_element_type=jnp.float32)
    h = jnp.maximum(h, 0.0)
    p2 = jnp.dot(h, Wl2[...], preferred_element_type=jnp.float32)
    b2 = (jnp.dot(h, Wr2[...], preferred_element_type=jnp.float32) + bl2[...])
    pb_o[...] = jnp.concatenate([p2, b2], axis=1)
    cnt_o[...] = cnt


def _tc_a(x1, msp, histsT, Wl1, bl1, Wr1, Wl2, bl2, Wr2):
    return pl.pallas_call(
        _tc_a_body,
        out_shape=[
            jax.ShapeDtypeStruct((N1, F), jnp.float32),
            jax.ShapeDtypeStruct((N1, 1), jnp.float32),
        ],
        interpret=_INTERPRET,
    )(x1, msp, histsT, Wl1.reshape(F, F), bl1.reshape(1, F), Wr1.reshape(F, F),
      Wl2.reshape(F, D2), bl2.reshape(1, D2), Wr2.reshape(F, D2))


def _tc_b_body(msum2, pb, cnt, src2, dst2, x2p,
               Wl1, bl1, Wr1, Wl2, bl2, Wr2, batch1,
               out1_o, out2_o, keys_o, counts_o):
    out1 = jnp.maximum(msum2[...] / cnt[...] + pb[:, D2:], 0.0)
    out1_o[...] = jnp.concatenate([out1, jnp.zeros_like(out1)], axis=1)

    cols = lax.broadcasted_iota(jnp.int32, (E2, N2P), 1)
    ohs = (cols == src2[...]).astype(jnp.float32)
    ohd = (cols == dst2[...]).astype(jnp.float32)
    A2 = jnp.dot(ohd.T, ohs, preferred_element_type=jnp.float32)
    cnt2 = jnp.maximum(jnp.sum(A2, axis=1, keepdims=True), 1.0)

    x2 = x2p[...]
    mean1 = jnp.dot(A2, x2, preferred_element_type=jnp.float32) / cnt2
    h2 = jnp.dot(mean1, Wl1[...], preferred_element_type=jnp.float32)
    h2 = h2 + bl1[...] + jnp.dot(x2, Wr1[...], preferred_element_type=jnp.float32)
    h2 = jnp.maximum(h2, 0.0)
    mean2 = jnp.dot(A2, h2, preferred_element_type=jnp.float32) / cnt2
    o2 = jnp.dot(mean2, Wl2[...], preferred_element_type=jnp.float32)
    o2 = o2 + bl2[...] + jnp.dot(h2, Wr2[...], preferred_element_type=jnp.float32)
    o2 = jnp.maximum(o2, 0.0)
    out2_o[...] = o2

    q = o2[N2 - 1:N2, :]
    sq1 = jnp.sum(out1 * out1, axis=1, keepdims=True)
    sqq = jnp.sum(q * q)
    d2 = sq1 + sqq - 2.0 * jnp.dot(out1, q.T, preferred_element_type=jnp.float32)
    keys_o[...] = jnp.sqrt(jnp.maximum(d2, 0.0) + 1e-12)

    bcols = lax.broadcasted_iota(jnp.int32, (N1, B), 1)
    counts_o[...] = jnp.sum((bcols == batch1[...]).astype(jnp.float32),
                            axis=0, keepdims=True)


def _tc_b(msum2, pb, cnt, src2, dst2, x2p, Wl1, bl1, Wr1, Wl2, bl2, Wr2, batch1):
    return pl.pallas_call(
        _tc_b_body,
        out_shape=[
            jax.ShapeDtypeStruct((N1, F), jnp.float32),
            jax.ShapeDtypeStruct((N2P, D2), jnp.float32),
            jax.ShapeDtypeStruct((N1, 1), jnp.float32),
            jax.ShapeDtypeStruct((1, B), jnp.float32),
        ],
        interpret=_INTERPRET,
    )(msum2, pb, cnt, src2, dst2, x2p,
      Wl1.reshape(F, F), bl1.reshape(1, F), Wr1.reshape(F, F),
      Wl2.reshape(F, D2), bl2.reshape(1, D2), Wr2.reshape(F, D2), batch1)


def _tc_c_body(sel, out2p, counts_col, W3p, fc1_b, g1, be1,
               fc2_w, fc2_b, g2, be2, fc3_wp, out_o):
    o2 = out2p[...]
    sq2 = jnp.sum(o2 * o2, axis=1)[None, :]
    o2t = o2.T
    cc = counts_col[...]

    acc = jnp.zeros((B, F), jnp.float32)
    for k in range(K):
        blk = sel[k * B:(k + 1) * B, :D2]
        sqs = jnp.sum(blk * blk, axis=1, keepdims=True)
        d2 = sqs + sq2 - 2.0 * jnp.dot(blk, o2t, preferred_element_type=jnp.float32)
        dist = jnp.sqrt(jnp.maximum(d2, 0.0) + 1e-12)
        dist = jnp.where(cc > k, dist, 0.0)
        acc = acc + jnp.dot(dist, W3p[k * N2P:(k + 1) * N2P, :],
                            preferred_element_type=jnp.float32)

    def _ln(v, g, be):
        mu = jnp.mean(v, axis=-1, keepdims=True)
        var = jnp.mean((v - mu) ** 2, axis=-1, keepdims=True)
        return (v - mu) / jnp.sqrt(var + 1e-5) * g + be

    h = jnp.maximum(_ln(acc + fc1_b[...], g1[...], be1[...]), 0.0)
    h = jnp.dot(h, fc2_w[...], preferred_element_type=jnp.float32) + fc2_b[...]
    h = jnp.maximum(_ln(h, g2[...], be2[...]), 0.0)
    res = jnp.dot(h, fc3_wp[...], preferred_element_type=jnp.float32)
    out_o[...] = jax.nn.sigmoid(res)


def _tc_c(sel, out2p, counts_col, W3p, fc1_b, g1, be1, fc2_w, fc2_b, g2, be2,
          fc3_wp):
    return pl.pallas_call(
        _tc_c_body,
        out_shape=jax.ShapeDtypeStruct((B, F), jnp.float32),
        interpret=_INTERPRET,
    )(sel, out2p, counts_col, W3p, fc1_b.reshape(1, F), g1.reshape(1, F),
      be1.reshape(1, F), fc2_w, fc2_b.reshape(1, D2), g2.reshape(1, D2),
      be2.reshape(1, D2), fc3_wp)


def kernel(x1, edge_index1, batch1, x2, edge_index2, Wl1, bl1, Wr1, Wl2, bl2,
           Wr2, fc1_w, fc1_b, g1, be1, fc2_w, fc2_b, g2, be2, fc3_w, fc3_b):
    src1, dst1 = edge_index1[0], edge_index1[1]

    hists = _sc_hist(dst1)
    msum1 = jax.ops.segment_sum(x1[src1], dst1, num_segments=N1)

    pb, cnt = _tc_a(x1, msum1, hists.T, Wl1, bl1, Wr1, Wl2, bl2, Wr2)

    msum2 = jax.ops.segment_sum(pb[edge_index1[0], :D2], edge_index1[1],
                                num_segments=N1)

    x2p = jnp.zeros((N2P, F), jnp.float32).at[:N2].set(x2)
    src2 = edge_index2[0].reshape(E2, 1)
    dst2 = edge_index2[1].reshape(E2, 1)
    out1, out2p, keys, counts = _tc_b(
        msum2, pb, cnt, src2, dst2, x2p, Wl1, bl1, Wr1, Wl2, bl2, Wr2,
        batch1.reshape(N1, 1))

    _USE_SC_TOPK = False
    if _USE_SC_TOPK:
        counts_i = counts.reshape(B).astype(jnp.int32)
        starts_i = jnp.concatenate(
            [jnp.zeros((1,), jnp.int32), jnp.cumsum(counts_i)[:-1]])
        sel = _sc_topk(keys.reshape(N1), batch1, starts_i, counts_i, out1)
    else:
        keys1 = keys[:, 0]
        binmask = batch1[None, :] == jnp.arange(B, dtype=batch1.dtype)[:, None]
        masked = jnp.where(binmask, keys1[None, :], -jnp.inf)
        _, topi = jax.lax.top_k(masked, K)
        idx_sm = topi.T.reshape(B * K)
        sel = jnp.zeros((KP * B, F), jnp.float32).at[:B * K].set(out1[idx_sm])

    W3p = jnp.zeros((K, N2P, F), jnp.float32).at[:, :N2, :].set(
        fc1_w.reshape(K, N2, F)).reshape(K * N2P, F)
    fc3_wp = jnp.zeros((D2, F), jnp.float32).at[:, :1].set(fc3_w)
    res = _tc_c(sel, out2p, counts.reshape(B, 1), W3p, fc1_b, g1, be1,
                fc2_w, fc2_b, g2, be2, fc3_wp)
    return res[:, :1]


# speedup vs baseline: 1.0208x; 1.0208x over previous
import functools
import jax
import jax.numpy as jnp
from jax import lax
from jax.experimental import pallas as pl
from jax.experimental.pallas import tpu as pltpu
from jax.experimental.pallas import tpu_sc as plsc

N1 = 10000
N2 = 199
N2P = 256
E2 = 3184
F = 128
B = 16
K = 50
D2 = 64

_INTERPRET = False

E1 = 640000
NC = 2
NS = 16
EC = 100
RPT = E1 // (NC * NS * EC)


def _sc_hist(dst_flat):
    mesh = plsc.VectorSubcoreMesh(core_axis_name="c", subcore_axis_name="s")
    EPT = E1 // (NC * NS)

    @functools.partial(
        pl.kernel, mesh=mesh,
        out_type=jax.ShapeDtypeStruct((NC * NS, N1), jnp.float32),
        scratch_types=[pltpu.VMEM((EPT,), jnp.int32),
                       pltpu.VMEM((N1,), jnp.float32)],
        compiler_params=pltpu.CompilerParams(needs_layout_passes=False))
    def k(df_h, hist_h, df_v, hist_v):
        c = lax.axis_index("c")
        s = lax.axis_index("s")
        w = c * NS + s
        pltpu.sync_copy(df_h.at[pl.ds(w * EPT, EPT)], df_v)
        zv = jnp.zeros((16,), jnp.float32)

        def zbody(i, carry):
            hist_v[pl.ds(i * 16, 16)] = zv
            return carry
        lax.fori_loop(0, N1 // 16, zbody, 0)

        def hbody(r, carry):
            d = df_v[pl.ds(r * 16, 16)]
            counts, islast = plsc.scan_count(d)
            plsc.addupdate_scatter(hist_v, [d], counts.astype(jnp.float32),
                                   mask=islast)
            return carry
        lax.fori_loop(0, EPT // 16, hbody, 0)
        pltpu.sync_copy(hist_v, hist_h.at[w])

    return k(dst_flat)


KP = 64
NEGF = -3.0e38
BIGI = 1 << 30


def _sc_topk(keys, batch, starts, counts, out1p):
    mesh = plsc.VectorSubcoreMesh(core_axis_name="c", subcore_axis_name="s")
    NV = N1 // 16

    @functools.partial(
        pl.kernel, mesh=mesh,
        out_type=jax.ShapeDtypeStruct((KP * B, F), jnp.float32),
        scratch_types=[
            pltpu.VMEM((N1,), jnp.float32),
            pltpu.VMEM((N1,), jnp.int32),
            pltpu.VMEM((16,), jnp.int32),
            pltpu.VMEM((16,), jnp.int32),
            pltpu.VMEM((KP,), jnp.int32),
            pltpu.VMEM((KP,), jnp.int32),
            pltpu.VMEM((KP, F), jnp.float32),
            pltpu.SemaphoreType.DMA,
            pltpu.SemaphoreType.DMA,
        ],
        compiler_params=pltpu.CompilerParams(needs_layout_passes=False),
    )
    def k(keys_h, batch_h, starts_h, counts_h, o1_h, sel_h,
          kb, bb, st_v, ct_v, win, ridx, rows, g0, s0):
        c = lax.axis_index("c")
        s = lax.axis_index("s")

        @pl.when(c >= 0)
        def _():
            pltpu.sync_copy(keys_h, kb)
            pltpu.sync_copy(batch_h, bb)
            pltpu.sync_copy(starts_h, st_v)
            pltpu.sync_copy(counts_h, ct_v)
            iota = lax.iota(jnp.int32, 16)
            lane0 = iota == 0
            svec = jnp.full((16,), s, jnp.int32)
            dnums = lax.GatherDimensionNumbers(
                offset_dims=(), collapsed_slice_dims=(0,),
                start_index_map=(0,))
            start_s = jnp.max(lax.gather(
                st_v[...], svec[:, None], dnums, (1,),
                mode=lax.GatherScatterMode.PROMISE_IN_BOUNDS))
            cnt_s = jnp.max(lax.gather(
                ct_v[...], svec[:, None], dnums, (1,),
                mode=lax.GatherScatterMode.PROMISE_IN_BOUNDS))
            jlo = start_s // 16
            jhi = (start_s + cnt_s + 15) // 16

            def mbody(j, carry):
                kv = kb[pl.ds(j * 16, 16)]
                bv = bb[pl.ds(j * 16, 16)]
                kb[pl.ds(j * 16, 16)] = jnp.where(bv == s, kv, NEGF)
                return carry
            lax.fori_loop(jlo, jhi, mbody, 0)

            for q in range(KP // 16):
                win[pl.ds(q * 16, 16)] = jnp.zeros((16,), jnp.int32)
                ridx[pl.ds(q * 16, 16)] = (iota + q * 16) * B + s

            def extract(kk, carry):
                def p1(j, m):
                    return jnp.maximum(m, kb[pl.ds(j * 16, 16)])
                mk = jnp.max(lax.fori_loop(
                    jlo, jhi, p1, jnp.full((16,), NEGF, jnp.float32)))

                def p2(j, mi):
                    kv = kb[pl.ds(j * 16, 16)]
                    cand = jnp.where(kv == mk, iota + j * 16, BIGI)
                    return jnp.minimum(mi, cand)
                wi = jnp.min(lax.fori_loop(
                    jlo, jhi, p2, jnp.full((16,), BIGI, jnp.int32)))
                wi = jnp.minimum(wi, N1 - 1)
                wv = jnp.full((16,), wi, jnp.int32)
                plsc.store_scatter(kb, [wv],
                                   jnp.full((16,), NEGF, jnp.float32),
                                   mask=lane0)
                plsc.store_scatter(win, [jnp.full((16,), kk, jnp.int32)],
                                   wv, mask=lane0)
                return carry
            lax.fori_loop(0, K, extract, 0)

            pltpu.async_copy(o1_h.at[win], rows, g0).wait()
            pltpu.async_copy(rows, sel_h.at[ridx], s0).wait()

    return k(keys, batch, starts, counts, out1p)


def _tc_a_body(x1, msum1, histsT, Wl1, bl1, Wr1, Wl2, bl2, Wr2, pb_o, cnt_o):
    cnt = jnp.maximum(jnp.sum(histsT[...], axis=1, keepdims=True), 1.0)
    mean = msum1[...] / cnt
    h = jnp.dot(mean, Wl1[...], preferred_element_type=jnp.float32)
    h = h + bl1[...] + jnp.dot(x1[...], Wr1[...], preferred_element_type=jnp.float32)
    h = jnp.maximum(h, 0.0)
    p2 = jnp.dot(h, Wl2[...], preferred_element_type=jnp.float32)
    b2 = (jnp.dot(h, Wr2[...], preferred_element_type=jnp.float32) + bl2[...])
    pb_o[...] = jnp.concatenate([p2, b2], axis=1)
    cnt_o[...] = cnt


def _tc_a(x1, msp, histsT, Wl1, bl1, Wr1, Wl2, bl2, Wr2):
    return pl.pallas_call(
        _tc_a_body,
        out_shape=[
            jax.ShapeDtypeStruct((N1, F), jnp.float32),
            jax.ShapeDtypeStruct((N1, 1), jnp.float32),
        ],
        interpret=_INTERPRET,
    )(x1, msp, histsT, Wl1.reshape(F, F), bl1.reshape(1, F), Wr1.reshape(F, F),
      Wl2.reshape(F, D2), bl2.reshape(1, D2), Wr2.reshape(F, D2))


def _tc_b_body(msum2, pb, cnt, src2, dst2, x2p,
               Wl1, bl1, Wr1, Wl2, bl2, Wr2, batch1,
               out1_o, out2_o, keys_o, counts_o):
    out1 = jnp.maximum(msum2[...] / cnt[...] + pb[:, D2:], 0.0)
    out1_o[...] = jnp.concatenate([out1, jnp.zeros_like(out1)], axis=1)

    cols = lax.broadcasted_iota(jnp.int32, (E2, N2P), 1)
    ohs = (cols == src2[...]).astype(jnp.float32)
    ohd = (cols == dst2[...]).astype(jnp.float32)
    A2 = jnp.dot(ohd.T, ohs, preferred_element_type=jnp.float32)
    cnt2 = jnp.maximum(jnp.sum(A2, axis=1, keepdims=True), 1.0)

    x2 = x2p[...]
    mean1 = jnp.dot(A2, x2, preferred_element_type=jnp.float32) / cnt2
    h2 = jnp.dot(mean1, Wl1[...], preferred_element_type=jnp.float32)
    h2 = h2 + bl1[...] + jnp.dot(x2, Wr1[...], preferred_element_type=jnp.float32)
    h2 = jnp.maximum(h2, 0.0)
    mean2 = jnp.dot(A2, h2, preferred_element_type=jnp.float32) / cnt2
    o2 = jnp.dot(mean2, Wl2[...], preferred_element_type=jnp.float32)
    o2 = o2 + bl2[...] + jnp.dot(h2, Wr2[...], preferred_element_type=jnp.float32)
    o2 = jnp.maximum(o2, 0.0)
    out2_o[...] = o2

    q = o2[N2 - 1:N2, :]
    sq1 = jnp.sum(out1 * out1, axis=1, keepdims=True)
    sqq = jnp.sum(q * q)
    d2 = sq1 + sqq - 2.0 * jnp.dot(out1, q.T, preferred_element_type=jnp.float32)
    keys_o[...] = jnp.sqrt(jnp.maximum(d2, 0.0) + 1e-12)

    bcols = lax.broadcasted_iota(jnp.int32, (N1, B), 1)
    counts_o[...] = jnp.sum((bcols == batch1[...]).astype(jnp.float32),
                            axis=0, keepdims=True)


def _tc_b(msum2, pb, cnt, src2, dst2, x2p, Wl1, bl1, Wr1, Wl2, bl2, Wr2, batch1):
    return pl.pallas_call(
        _tc_b_body,
        out_shape=[
            jax.ShapeDtypeStruct((N1, F), jnp.float32),
            jax.ShapeDtypeStruct((N2P, D2), jnp.float32),
            jax.ShapeDtypeStruct((N1, 1), jnp.float32),
            jax.ShapeDtypeStruct((1, B), jnp.float32),
        ],
        interpret=_INTERPRET,
    )(msum2, pb, cnt, src2, dst2, x2p,
      Wl1.reshape(F, F), bl1.reshape(1, F), Wr1.reshape(F, F),
      Wl2.reshape(F, D2), bl2.reshape(1, D2), Wr2.reshape(F, D2), batch1)


def _tc_c_body(sel, out2p, counts_col, W3p, fc1_b, g1, be1,
               fc2_w, fc2_b, g2, be2, fc3_wp, out_o):
    o2 = out2p[...]
    sq2 = jnp.sum(o2 * o2, axis=1)[None, :]
    o2t = o2.T
    cc = counts_col[...]

    acc = jnp.zeros((B, F), jnp.float32)
    for k in range(K):
        blk = sel[k * B:(k + 1) * B, :D2]
        sqs = jnp.sum(blk * blk, axis=1, keepdims=True)
        d2 = sqs + sq2 - 2.0 * jnp.dot(blk, o2t, preferred_element_type=jnp.float32)
        dist = jnp.sqrt(jnp.maximum(d2, 0.0) + 1e-12)
        dist = jnp.where(cc > k, dist, 0.0)
        acc = acc + jnp.dot(dist, W3p[k * N2P:(k + 1) * N2P, :],
                            preferred_element_type=jnp.float32)

    def _ln(v, g, be):
        mu = jnp.mean(v, axis=-1, keepdims=True)
        var = jnp.mean((v - mu) ** 2, axis=-1, keepdims=True)
        return (v - mu) / jnp.sqrt(var + 1e-5) * g + be

    h = jnp.maximum(_ln(acc + fc1_b[...], g1[...], be1[...]), 0.0)
    h = jnp.dot(h, fc2_w[...], preferred_element_type=jnp.float32) + fc2_b[...]
    h = jnp.maximum(_ln(h, g2[...], be2[...]), 0.0)
    res = jnp.dot(h, fc3_wp[...], preferred_element_type=jnp.float32)
    out_o[...] = jax.nn.sigmoid(res)


def _tc_c(sel, out2p, counts_col, W3p, fc1_b, g1, be1, fc2_w, fc2_b, g2, be2,
          fc3_wp):
    return pl.pallas_call(
        _tc_c_body,
        out_shape=jax.ShapeDtypeStruct((B, F), jnp.float32),
        interpret=_INTERPRET,
    )(sel, out2p, counts_col, W3p, fc1_b.reshape(1, F), g1.reshape(1, F),
      be1.reshape(1, F), fc2_w, fc2_b.reshape(1, D2), g2.reshape(1, D2),
      be2.reshape(1, D2), fc3_wp)


def kernel(x1, edge_index1, batch1, x2, edge_index2, Wl1, bl1, Wr1, Wl2, bl2,
           Wr2, fc1_w, fc1_b, g1, be1, fc2_w, fc2_b, g2, be2, fc3_w, fc3_b):
    src1, dst1 = edge_index1[0], edge_index1[1]

    hists = jnp.zeros((NC * NS, N1), jnp.float32).at[0].set(jnp.bincount(dst1, length=N1).astype(jnp.float32))
    msum1 = jax.ops.segment_sum(x1[src1], dst1, num_segments=N1)

    pb, cnt = _tc_a(x1, msum1, hists.T, Wl1, bl1, Wr1, Wl2, bl2, Wr2)

    msum2 = jax.ops.segment_sum(pb[edge_index1[0], :D2], edge_index1[1],
                                num_segments=N1)

    x2p = jnp.zeros((N2P, F), jnp.float32).at[:N2].set(x2)
    src2 = edge_index2[0].reshape(E2, 1)
    dst2 = edge_index2[1].reshape(E2, 1)
    out1, out2p, keys, counts = _tc_b(
        msum2, pb, cnt, src2, dst2, x2p, Wl1, bl1, Wr1, Wl2, bl2, Wr2,
        batch1.reshape(N1, 1))

    _USE_SC_TOPK = True
    if _USE_SC_TOPK:
        counts_i = counts.reshape(B).astype(jnp.int32)
        starts_i = jnp.concatenate(
            [jnp.zeros((1,), jnp.int32), jnp.cumsum(counts_i)[:-1]])
        sel = _sc_topk(keys.reshape(N1), batch1, starts_i, counts_i, out1)
    else:
        keys1 = keys[:, 0]
        binmask = batch1[None, :] == jnp.arange(B, dtype=batch1.dtype)[:, None]
        masked = jnp.where(binmask, keys1[None, :], -jnp.inf)
        _, topi = jax.lax.top_k(masked, K)
        idx_sm = topi.T.reshape(B * K)
        sel = jnp.zeros((KP * B, F), jnp.float32).at[:B * K].set(out1[idx_sm])

    W3p = jnp.zeros((K, N2P, F), jnp.float32).at[:, :N2, :].set(
        fc1_w.reshape(K, N2, F)).reshape(K * N2P, F)
    fc3_wp = jnp.zeros((D2, F), jnp.float32).at[:, :1].set(fc3_w)
    res = _tc_c(sel, out2p, counts.reshape(B, 1), W3p, fc1_b, g1, be1,
                fc2_w, fc2_b, g2, be2, fc3_wp)
    return res[:, :1]


# speedup vs baseline: 2.6314x; 2.5778x over previous
import functools
import jax
import jax.numpy as jnp
from jax import lax
from jax.experimental import pallas as pl
from jax.experimental.pallas import tpu as pltpu
from jax.experimental.pallas import tpu_sc as plsc

N1 = 10000
N2 = 199
N2P = 256
E2 = 3184
F = 128
B = 16
K = 50
D2 = 64

_INTERPRET = False

E1 = 640000
NC = 2
NS = 16
EC = 100
RPT = E1 // (NC * NS * EC)


def _sc_hist(dst_flat):
    mesh = plsc.VectorSubcoreMesh(core_axis_name="c", subcore_axis_name="s")
    EPT = E1 // (NC * NS)

    @functools.partial(
        pl.kernel, mesh=mesh,
        out_type=jax.ShapeDtypeStruct((NC * NS, N1), jnp.float32),
        scratch_types=[pltpu.VMEM((EPT,), jnp.int32),
                       pltpu.VMEM((N1,), jnp.float32)],
        compiler_params=pltpu.CompilerParams(needs_layout_passes=False))
    def k(df_h, hist_h, df_v, hist_v):
        c = lax.axis_index("c")
        s = lax.axis_index("s")
        w = c * NS + s
        pltpu.sync_copy(df_h.at[pl.ds(w * EPT, EPT)], df_v)
        zv = jnp.zeros((16,), jnp.float32)

        def zbody(i, carry):
            hist_v[pl.ds(i * 16, 16)] = zv
            return carry
        lax.fori_loop(0, N1 // 16, zbody, 0)

        def hbody(r, carry):
            d = df_v[pl.ds(r * 16, 16)]
            counts, islast = plsc.scan_count(d)
            plsc.addupdate_scatter(hist_v, [d], counts.astype(jnp.float32),
                                   mask=islast)
            return carry
        lax.fori_loop(0, EPT // 16, hbody, 0)
        pltpu.sync_copy(hist_v, hist_h.at[w])

    return k(dst_flat)


KP = 64
NEGF = -3.0e38
BIGI = 1 << 30


def _sc_topk(keys, batch, starts, counts, out1p):
    mesh = plsc.VectorSubcoreMesh(core_axis_name="c", subcore_axis_name="s")
    NV = N1 // 16

    @functools.partial(
        pl.kernel, mesh=mesh,
        out_type=jax.ShapeDtypeStruct((KP * B, F), jnp.float32),
        scratch_types=[
            pltpu.VMEM((N1,), jnp.float32),
            pltpu.VMEM((N1,), jnp.int32),
            pltpu.VMEM((16,), jnp.int32),
            pltpu.VMEM((16,), jnp.int32),
            pltpu.VMEM((KP,), jnp.int32),
            pltpu.VMEM((KP,), jnp.int32),
            pltpu.VMEM((KP, F), jnp.float32),
            pltpu.SemaphoreType.DMA,
            pltpu.SemaphoreType.DMA,
        ],
        compiler_params=pltpu.CompilerParams(needs_layout_passes=False),
    )
    def k(keys_h, batch_h, starts_h, counts_h, o1_h, sel_h,
          kb, bb, st_v, ct_v, win, ridx, rows, g0, s0):
        c = lax.axis_index("c")
        s = lax.axis_index("s")

        @pl.when(c >= 0)
        def _():
            pltpu.sync_copy(keys_h, kb)
            pltpu.sync_copy(batch_h, bb)
            pltpu.sync_copy(starts_h, st_v)
            pltpu.sync_copy(counts_h, ct_v)
            iota = lax.iota(jnp.int32, 16)
            lane0 = iota == 0
            svec = jnp.full((16,), s, jnp.int32)
            dnums = lax.GatherDimensionNumbers(
                offset_dims=(), collapsed_slice_dims=(0,),
                start_index_map=(0,))
            start_s = jnp.max(lax.gather(
                st_v[...], svec[:, None], dnums, (1,),
                mode=lax.GatherScatterMode.PROMISE_IN_BOUNDS))
            cnt_s = jnp.max(lax.gather(
                ct_v[...], svec[:, None], dnums, (1,),
                mode=lax.GatherScatterMode.PROMISE_IN_BOUNDS))
            jlo = start_s // 16
            jhi = (start_s + cnt_s + 15) // 16

            def mbody(j, carry):
                kv = kb[pl.ds(j * 16, 16)]
                bv = bb[pl.ds(j * 16, 16)]
                kb[pl.ds(j * 16, 16)] = jnp.where(bv == s, kv, NEGF)
                return carry
            lax.fori_loop(jlo, jhi, mbody, 0)

            for q in range(KP // 16):
                win[pl.ds(q * 16, 16)] = jnp.zeros((16,), jnp.int32)
                ridx[pl.ds(q * 16, 16)] = (iota + q * 16) * B + s

            def extract(kk, carry):
                def p1(j, m):
                    return jnp.maximum(m, kb[pl.ds(j * 16, 16)])
                mk = jnp.max(lax.fori_loop(
                    jlo, jhi, p1, jnp.full((16,), NEGF, jnp.float32)))

                def p2(j, mi):
                    kv = kb[pl.ds(j * 16, 16)]
                    cand = jnp.where(kv == mk, iota + j * 16, BIGI)
                    return jnp.minimum(mi, cand)
                wi = jnp.min(lax.fori_loop(
                    jlo, jhi, p2, jnp.full((16,), BIGI, jnp.int32)))
                wi = jnp.minimum(wi, N1 - 1)
                wv = jnp.full((16,), wi, jnp.int32)
                plsc.store_scatter(kb, [wv],
                                   jnp.full((16,), NEGF, jnp.float32),
                                   mask=lane0)
                plsc.store_scatter(win, [jnp.full((16,), kk, jnp.int32)],
                                   wv, mask=lane0)
                return carry
            lax.fori_loop(0, K, extract, 0)

            pltpu.async_copy(o1_h.at[win], rows, g0).wait()
            pltpu.async_copy(rows, sel_h.at[ridx], s0).wait()

    return k(keys, batch, starts, counts, out1p)


def _tc_a_body(x1, msum1, histsT, Wl1, bl1, Wr1, Wl2, bl2, Wr2, pb_o, cnt_o):
    cnt = jnp.maximum(jnp.sum(histsT[...], axis=1, keepdims=True), 1.0)
    mean = msum1[...] / cnt
    h = jnp.dot(mean, Wl1[...], preferred_element_type=jnp.float32)
    h = h + bl1[...] + jnp.dot(x1[...], Wr1[...], preferred_element_type=jnp.float32)
    h = jnp.maximum(h, 0.0)
    p2 = jnp.dot(h, Wl2[...], preferred_element_type=jnp.float32)
    b2 = (jnp.dot(h, Wr2[...], preferred_element_type=jnp.float32) + bl2[...])
    pb_o[...] = jnp.concatenate([p2, b2], axis=1)
    cnt_o[...] = cnt


def _tc_a(x1, msp, histsT, Wl1, bl1, Wr1, Wl2, bl2, Wr2):
    return pl.pallas_call(
        _tc_a_body,
        out_shape=[
            jax.ShapeDtypeStruct((N1, F), jnp.float32),
            jax.ShapeDtypeStruct((N1, 1), jnp.float32),
        ],
        interpret=_INTERPRET,
    )(x1, msp, histsT, Wl1.reshape(F, F), bl1.reshape(1, F), Wr1.reshape(F, F),
      Wl2.reshape(F, D2), bl2.reshape(1, D2), Wr2.reshape(F, D2))


def _tc_b_body(msum2, pb, cnt, src2, dst2, x2p,
               Wl1, bl1, Wr1, Wl2, bl2, Wr2, batch1,
               out1_o, out2_o, keys_o, counts_o):
    out1 = jnp.maximum(msum2[...] / cnt[...] + pb[:, D2:], 0.0)
    out1_o[...] = jnp.concatenate([out1, jnp.zeros_like(out1)], axis=1)

    cols = lax.broadcasted_iota(jnp.int32, (E2, N2P), 1)
    ohs = (cols == src2[...]).astype(jnp.float32)
    ohd = (cols == dst2[...]).astype(jnp.float32)
    A2 = jnp.dot(ohd.T, ohs, preferred_element_type=jnp.float32)
    cnt2 = jnp.maximum(jnp.sum(A2, axis=1, keepdims=True), 1.0)

    x2 = x2p[...]
    mean1 = jnp.dot(A2, x2, preferred_element_type=jnp.float32) / cnt2
    h2 = jnp.dot(mean1, Wl1[...], preferred_element_type=jnp.float32)
    h2 = h2 + bl1[...] + jnp.dot(x2, Wr1[...], preferred_element_type=jnp.float32)
    h2 = jnp.maximum(h2, 0.0)
    mean2 = jnp.dot(A2, h2, preferred_element_type=jnp.float32) / cnt2
    o2 = jnp.dot(mean2, Wl2[...], preferred_element_type=jnp.float32)
    o2 = o2 + bl2[...] + jnp.dot(h2, Wr2[...], preferred_element_type=jnp.float32)
    o2 = jnp.maximum(o2, 0.0)
    out2_o[...] = o2

    q = o2[N2 - 1:N2, :]
    sq1 = jnp.sum(out1 * out1, axis=1, keepdims=True)
    sqq = jnp.sum(q * q)
    d2 = sq1 + sqq - 2.0 * jnp.dot(out1, q.T, preferred_element_type=jnp.float32)
    keys_o[...] = jnp.sqrt(jnp.maximum(d2, 0.0) + 1e-12)

    bcols = lax.broadcasted_iota(jnp.int32, (N1, B), 1)
    counts_o[...] = jnp.sum((bcols == batch1[...]).astype(jnp.float32),
                            axis=0, keepdims=True)


def _tc_b(msum2, pb, cnt, src2, dst2, x2p, Wl1, bl1, Wr1, Wl2, bl2, Wr2, batch1):
    return pl.pallas_call(
        _tc_b_body,
        out_shape=[
            jax.ShapeDtypeStruct((N1, F), jnp.float32),
            jax.ShapeDtypeStruct((N2P, D2), jnp.float32),
            jax.ShapeDtypeStruct((N1, 1), jnp.float32),
            jax.ShapeDtypeStruct((1, B), jnp.float32),
        ],
        interpret=_INTERPRET,
    )(msum2, pb, cnt, src2, dst2, x2p,
      Wl1.reshape(F, F), bl1.reshape(1, F), Wr1.reshape(F, F),
      Wl2.reshape(F, D2), bl2.reshape(1, D2), Wr2.reshape(F, D2), batch1)


def _tc_c_body(sel, out2p, counts_col, W3p, fc1_b, g1, be1,
               fc2_w, fc2_b, g2, be2, fc3_wp, out_o):
    o2 = out2p[...]
    sq2 = jnp.sum(o2 * o2, axis=1)[None, :]
    o2t = o2.T
    cc = counts_col[...]

    acc = jnp.zeros((B, F), jnp.float32)
    for k in range(K):
        blk = sel[k * B:(k + 1) * B, :D2]
        sqs = jnp.sum(blk * blk, axis=1, keepdims=True)
        d2 = sqs + sq2 - 2.0 * jnp.dot(blk, o2t, preferred_element_type=jnp.float32)
        dist = jnp.sqrt(jnp.maximum(d2, 0.0) + 1e-12)
        dist = jnp.where(cc > k, dist, 0.0)
        acc = acc + jnp.dot(dist, W3p[k * N2P:(k + 1) * N2P, :],
                            preferred_element_type=jnp.float32)

    def _ln(v, g, be):
        mu = jnp.mean(v, axis=-1, keepdims=True)
        var = jnp.mean((v - mu) ** 2, axis=-1, keepdims=True)
        return (v - mu) / jnp.sqrt(var + 1e-5) * g + be

    h = jnp.maximum(_ln(acc + fc1_b[...], g1[...], be1[...]), 0.0)
    h = jnp.dot(h, fc2_w[...], preferred_element_type=jnp.float32) + fc2_b[...]
    h = jnp.maximum(_ln(h, g2[...], be2[...]), 0.0)
    res = jnp.dot(h, fc3_wp[...], preferred_element_type=jnp.float32)
    out_o[...] = jax.nn.sigmoid(res)


def _tc_c(sel, out2p, counts_col, W3p, fc1_b, g1, be1, fc2_w, fc2_b, g2, be2,
          fc3_wp):
    return pl.pallas_call(
        _tc_c_body,
        out_shape=jax.ShapeDtypeStruct((B, F), jnp.float32),
        interpret=_INTERPRET,
    )(sel, out2p, counts_col, W3p, fc1_b.reshape(1, F), g1.reshape(1, F),
      be1.reshape(1, F), fc2_w, fc2_b.reshape(1, D2), g2.reshape(1, D2),
      be2.reshape(1, D2), fc3_wp)


def kernel(x1, edge_index1, batch1, x2, edge_index2, Wl1, bl1, Wr1, Wl2, bl2,
           Wr2, fc1_w, fc1_b, g1, be1, fc2_w, fc2_b, g2, be2, fc3_w, fc3_b):
    src1, dst1 = edge_index1[0], edge_index1[1]

    hists = jnp.zeros((NC * NS, N1), jnp.float32).at[0].set(jnp.bincount(dst1, length=N1).astype(jnp.float32))
    msum1 = jax.ops.segment_sum(x1[src1], dst1, num_segments=N1)

    pb, cnt = _tc_a(x1, msum1, hists.T, Wl1, bl1, Wr1, Wl2, bl2, Wr2)

    msum2 = jax.ops.segment_sum(pb[edge_index1[0], :D2], edge_index1[1],
                                num_segments=N1)

    x2p = jnp.zeros((N2P, F), jnp.float32).at[:N2].set(x2)
    src2 = edge_index2[0].reshape(E2, 1)
    dst2 = edge_index2[1].reshape(E2, 1)
    out1, out2p, keys, counts = _tc_b(
        msum2, pb, cnt, src2, dst2, x2p, Wl1, bl1, Wr1, Wl2, bl2, Wr2,
        batch1.reshape(N1, 1))

    _USE_SC_TOPK = False
    if _USE_SC_TOPK:
        counts_i = counts.reshape(B).astype(jnp.int32)
        starts_i = jnp.concatenate(
            [jnp.zeros((1,), jnp.int32), jnp.cumsum(counts_i)[:-1]])
        sel = _sc_topk(keys.reshape(N1), batch1, starts_i, counts_i, out1)
    else:
        keys1 = keys[:, 0]
        binmask = batch1[None, :] == jnp.arange(B, dtype=batch1.dtype)[:, None]
        masked = jnp.where(binmask, keys1[None, :], -jnp.inf)
        _, topi = jax.lax.top_k(masked, K)
        idx_sm = topi.T.reshape(B * K)
        sel = jnp.zeros((KP * B, F), jnp.float32).at[:B * K].set(out1[idx_sm])

    W3p = jnp.zeros((K, N2P, F), jnp.float32).at[:, :N2, :].set(
        fc1_w.reshape(K, N2, F)).reshape(K * N2P, F)
    fc3_wp = jnp.zeros((D2, F), jnp.float32).at[:, :1].set(fc3_w)
    res = _tc_c(sel, out2p, counts.reshape(B, 1), W3p, fc1_b, g1, be1,
                fc2_w, fc2_b, g2, be2, fc3_wp)
    return res[:, :1]
